# Initial kernel scaffold; baseline (speedup 1.0000x reference)
import jax
import jax.numpy as jnp
from jax.experimental import pallas as pl


def kernel(features, edge_index, W0, al0, ar0, b0, W1, al1, ar1, b1, W2, al2, ar2, b2):
    raise NotImplementedError("write your pallas kernel here")


# speedup vs baseline: 28.3545x; 28.3545x over previous
import functools

import jax
import jax.numpy as jnp
from jax import lax
from jax.experimental import pallas as pl
from jax.experimental.pallas import tpu as pltpu
from jax.experimental.pallas import tpu_sc as plsc

_N = 10000
_E = 320000
_H = 8
_DH = 16
_NCLS = 40
_DP = 64

_NCORES = 2
_NSUB = 16
_NW = _NCORES * _NSUB
_EPW = _E // _NW
_CH = 80
_NCHUNK = _EPW // _CH
_ZR = 200
_NZB = _N // _ZR


def _first_body(x_ref, w_ref, alr_ref, hlo_ref, hhi_ref, elr_ref):
    h = jnp.dot(x_ref[...], w_ref[...], preferred_element_type=jnp.float32)
    hlo_ref[...] = h[:, :_DP]
    hhi_ref[...] = h[:, _DP:]
    elr_ref[...] = jnp.dot(h, alr_ref[...], preferred_element_type=jnp.float32)


def _make_mid_body(two_halves):
    def _mid_body(olo0_ref, olo1_ref, ohi0_ref, ohi1_ref, slo0_ref, slo1_ref,
                  shi0_ref, shi1_ref, e4_ref, blo_ref, bhi_ref, wlo_ref,
                  whi_ref, alr_ref, *out_refs):
        s_lo = (slo0_ref[...] + slo1_ref[...])[:, :4]
        s_hi = (shi0_ref[...] + shi1_ref[...])[:, :4]
        e4 = e4_ref[...]
        d_lo = jnp.dot(s_lo, e4, preferred_element_type=jnp.float32) + 1e-9
        d_hi = jnp.dot(s_hi, e4, preferred_element_type=jnp.float32) + 1e-9
        x_lo = (olo0_ref[...] + olo1_ref[...]) / d_lo + blo_ref[...]
        x_hi = (ohi0_ref[...] + ohi1_ref[...]) / d_hi + bhi_ref[...]
        x_lo = jnp.maximum(x_lo, 0.0)
        x_hi = jnp.maximum(x_hi, 0.0)
        h = (jnp.dot(x_lo, wlo_ref[...], preferred_element_type=jnp.float32)
             + jnp.dot(x_hi, whi_ref[...], preferred_element_type=jnp.float32))
        if two_halves:
            hlo_ref, hhi_ref, elr_ref = out_refs
            hlo_ref[...] = h[:, :_DP]
            hhi_ref[...] = h[:, _DP:]
        else:
            hlo_ref, elr_ref = out_refs
            hlo_ref[...] = h
        elr_ref[...] = jnp.dot(h, alr_ref[...],
                               preferred_element_type=jnp.float32)
    return _mid_body


def _fin_body(o0_ref, o1_ref, s0_ref, s1_ref, b_ref, out_ref):
    o = o0_ref[...] + o1_ref[...]
    s = (s0_ref[...] + s1_ref[...])[:, 0:1]
    out_ref[...] = o[:, :_NCLS] / (s + 1e-9) + b_ref[...]


_BM = 400


def _first(x, W, ALR):
    return pl.pallas_call(
        _first_body,
        grid=(_N // _BM,),
        in_specs=[pl.BlockSpec((_BM, 128), lambda i: (i, 0)),
                  pl.BlockSpec((128, 128), lambda i: (0, 0)),
                  pl.BlockSpec((128, 16), lambda i: (0, 0))],
        out_specs=[pl.BlockSpec((_BM, _DP), lambda i: (i, 0)),
                   pl.BlockSpec((_BM, _DP), lambda i: (i, 0)),
                   pl.BlockSpec((_BM, 16), lambda i: (i, 0))],
        out_shape=[jax.ShapeDtypeStruct((_N, _DP), jnp.float32),
                   jax.ShapeDtypeStruct((_N, _DP), jnp.float32),
                   jax.ShapeDtypeStruct((_N, 16), jnp.float32)],
    )(x, W, ALR)


def _mid(olo0, olo1, ohi0, ohi1, slo0, slo1, shi0, shi1, E4, b_lo, b_hi,
         W_lo, W_hi, ALR, dp_out):
    two = dp_out == 128
    out_specs = [pl.BlockSpec((_BM, _DP), lambda i: (i, 0))]
    out_shape = [jax.ShapeDtypeStruct((_N, _DP), jnp.float32)]
    if two:
        out_specs.append(pl.BlockSpec((_BM, _DP), lambda i: (i, 0)))
        out_shape.append(jax.ShapeDtypeStruct((_N, _DP), jnp.float32))
    out_specs.append(pl.BlockSpec((_BM, 16), lambda i: (i, 0)))
    out_shape.append(jax.ShapeDtypeStruct((_N, 16), jnp.float32))
    return pl.pallas_call(
        _make_mid_body(two),
        grid=(_N // _BM,),
        in_specs=[pl.BlockSpec((_BM, _DP), lambda i: (i, 0)),
                  pl.BlockSpec((_BM, _DP), lambda i: (i, 0)),
                  pl.BlockSpec((_BM, _DP), lambda i: (i, 0)),
                  pl.BlockSpec((_BM, _DP), lambda i: (i, 0)),
                  pl.BlockSpec((_BM, 16), lambda i: (i, 0)),
                  pl.BlockSpec((_BM, 16), lambda i: (i, 0)),
                  pl.BlockSpec((_BM, 16), lambda i: (i, 0)),
                  pl.BlockSpec((_BM, 16), lambda i: (i, 0)),
                  pl.BlockSpec((4, _DP), lambda i: (0, 0)),
                  pl.BlockSpec((1, _DP), lambda i: (0, 0)),
                  pl.BlockSpec((1, _DP), lambda i: (0, 0)),
                  pl.BlockSpec((_DP, dp_out), lambda i: (0, 0)),
                  pl.BlockSpec((_DP, dp_out), lambda i: (0, 0)),
                  pl.BlockSpec((dp_out, 16), lambda i: (0, 0))],
        out_specs=out_specs,
        out_shape=out_shape,
    )(olo0, olo1, ohi0, ohi1, slo0, slo1, shi0, shi1, E4, b_lo, b_hi,
      W_lo, W_hi, ALR)


def _fin(o0, o1, s0, s1, b):
    return pl.pallas_call(
        _fin_body,
        grid=(_N // _BM,),
        in_specs=[pl.BlockSpec((_BM, _DP), lambda i: (i, 0)),
                  pl.BlockSpec((_BM, _DP), lambda i: (i, 0)),
                  pl.BlockSpec((_BM, 16), lambda i: (i, 0)),
                  pl.BlockSpec((_BM, 16), lambda i: (i, 0)),
                  pl.BlockSpec((1, _NCLS), lambda i: (0, 0))],
        out_specs=pl.BlockSpec((_BM, _NCLS), lambda i: (i, 0)),
        out_shape=jax.ShapeDtypeStruct((_N, _NCLS), jnp.float32),
    )(o0, o1, s0, s1, b)


_mesh = plsc.VectorSubcoreMesh(core_axis_name="c", subcore_axis_name="s")


@functools.partial(
    pl.kernel,
    mesh=_mesh,
    compiler_params=pltpu.CompilerParams(needs_layout_passes=False,
                                         use_tc_tiling_on_sc=False),
    out_type=[jax.ShapeDtypeStruct((_NCORES, _N, _DP), jnp.float32),
              jax.ShapeDtypeStruct((_NCORES, _N, 16), jnp.float32)],
    scratch_types=[
        pltpu.VMEM((16,), jnp.int32),
        pltpu.VMEM((_CH,), jnp.int32),
        pltpu.VMEM((_CH,), jnp.int32),
        pltpu.VMEM((_CH, 16), jnp.float32),
        pltpu.VMEM((_CH, 16), jnp.float32),
        pltpu.VMEM((_CH, 16), jnp.float32),
        pltpu.VMEM((_CH, _DP), jnp.float32),
        pltpu.VMEM((_ZR, _DP), jnp.float32),
        pltpu.VMEM((_ZR, 16), jnp.float32),
        pltpu.VMEM_SHARED((_N, _DP), jnp.float32),
        pltpu.VMEM_SHARED((_N, 16), jnp.float32),
        pltpu.SemaphoreType.DMA,
    ],
)
def _edge_pass(h_hbm, elr_hbm, src_hbm, dst_hbm, hoff_hbm, o_out, s_out,
               hoffb, sidx, didx, elrs, elrd, eebuf, hbuf, zbuf, zbufs,
               o_acc, s_acc, sem):
    cid = lax.axis_index("c")
    sid = lax.axis_index("s")
    wid = sid * _NCORES + cid
    zero16 = jnp.zeros((16,), jnp.float32)

    pltpu.sync_copy(hoff_hbm, hoffb)
    hv = hoffb[:]

    def zrow(i, carry):
        for v in range(_DP // 16):
            zbuf[i, pl.ds(16 * v, 16)] = zero16
        zbufs[i, :] = zero16
        return carry
    lax.fori_loop(0, _ZR, zrow, 0)

    def zee(i, carry):
        eebuf[i, :] = zero16
        return carry
    lax.fori_loop(0, _CH, zee, 0)

    for k in range((_NZB + _NSUB - 1) // _NSUB):
        blk = sid + _NSUB * k
        @pl.when(blk < _NZB)
        def _():
            r0 = blk * _ZR
            pltpu.sync_copy(zbuf, o_acc.at[pl.ds(r0, _ZR)])
            pltpu.sync_copy(zbufs, s_acc.at[pl.ds(r0, _ZR)])
    plsc.subcore_barrier()

    iot = lax.iota(jnp.int32, 16)
    base = wid * _EPW

    def chunk_body(g, carry):
        off = base + g * _CH
        pltpu.sync_copy(src_hbm.at[pl.ds(off, _CH)], sidx)
        pltpu.sync_copy(dst_hbm.at[pl.ds(off, _CH)], didx)
        cp = pltpu.async_copy(h_hbm.at[sidx], hbuf, sem)
        pltpu.sync_copy(elr_hbm.at[sidx], elrs)
        pltpu.sync_copy(elr_hbm.at[didx], elrd)
        for t in range(_CH // 16):
            rows = iot + (16 * t)
            for j in range(4):
                el16 = plsc.load_gather(elrs, [rows, hv + j])
                er16 = plsc.load_gather(elrd, [rows, hv + (8 + j)])
                e = el16 + er16
                e = jnp.where(e >= 0.0, e, 0.2 * e)
                plsc.store_scatter(eebuf, [rows, jnp.full((16,), j, jnp.int32)],
                                   jnp.exp(e))
        pltpu.sync_copy(eebuf, s_acc.at[didx], add=True)
        cp.wait()

        def medge(i, c2):
            iv = jnp.full((16,), i, jnp.int32)
            for v in range(_DP // 16):
                w = plsc.load_gather(eebuf, [iv, jnp.full((16,), v, jnp.int32)])
                hval = hbuf[i, pl.ds(16 * v, 16)]
                hbuf[i, pl.ds(16 * v, 16)] = hval * w
            return c2
        lax.fori_loop(0, _CH, medge, 0)
        pltpu.sync_copy(hbuf, o_acc.at[didx], add=True)
        return carry
    lax.fori_loop(0, _NCHUNK, chunk_body, 0)

    plsc.subcore_barrier()
    for k in range((_NZB + _NSUB - 1) // _NSUB):
        blk = sid + _NSUB * k
        @pl.when(blk < _NZB)
        def _():
            r0 = blk * _ZR
            pltpu.sync_copy(o_acc.at[pl.ds(r0, _ZR)],
                            o_out.at[cid, pl.ds(r0, _ZR)])
            pltpu.sync_copy(s_acc.at[pl.ds(r0, _ZR)],
                            s_out.at[cid, pl.ds(r0, _ZR)])


def _alr_mat(al, ar, heads, dout):
    alf = al.reshape(-1)
    arf = ar.reshape(-1)
    d = heads * dout
    mask = (jnp.arange(d)[:, None] // dout == jnp.arange(heads)[None, :])
    mask = mask.astype(jnp.float32)
    ALm = mask * alf[:, None]
    ARm = mask * arf[:, None]
    pad = jnp.zeros((d, 8 - heads), jnp.float32)
    return jnp.concatenate([ALm, pad, ARm, pad], axis=1)


def kernel(features, edge_index, W0, al0, ar0, b0, W1, al1, ar1, b1,
           W2, al2, ar2, b2):
    src = edge_index[0]
    dst = edge_index[1]

    ALR0 = _alr_mat(al0, ar0, _H, _DH)
    ALR1 = _alr_mat(al1, ar1, _H, _DH)
    al2f = jnp.pad(al2.reshape(-1), (0, _DP - _NCLS))
    ar2f = jnp.pad(ar2.reshape(-1), (0, _DP - _NCLS))
    rep = (jnp.arange(16)[None, :] < 4).astype(jnp.float32)
    rep_r = ((jnp.arange(16)[None, :] >= 8)
             & (jnp.arange(16)[None, :] < 12)).astype(jnp.float32)
    ALR2 = al2f[:, None] * rep + ar2f[:, None] * rep_r
    W2p = jnp.pad(W2, ((0, 0), (0, _DP - _NCLS)))
    E4 = (jnp.arange(_DP)[None, :] // 16 == jnp.arange(4)[:, None])
    E4 = E4.astype(jnp.float32)
    b0f = b0.reshape(1, 128)
    b1f = b1.reshape(1, 128)
    b2f = b2.reshape(1, _NCLS)
    hoff0 = jnp.zeros((16,), jnp.int32)
    hoff4 = jnp.full((16,), 4, jnp.int32)

    def run_layer(h_lo, h_hi, elr):
        olo, slo = _edge_pass(h_lo, elr, src, dst, hoff0)
        ohi, shi = _edge_pass(h_hi, elr, src, dst, hoff4)
        return olo, ohi, slo, shi

    h_lo, h_hi, elr = _first(features, W0, ALR0)
    olo, ohi, slo, shi = run_layer(h_lo, h_hi, elr)
    h_lo, h_hi, elr = _mid(olo[0], olo[1], ohi[0], ohi[1], slo[0], slo[1],
                           shi[0], shi[1], E4, b0f[:, :_DP], b0f[:, _DP:],
                           W1[:_DP], W1[_DP:], ALR1, 128)
    olo, ohi, slo, shi = run_layer(h_lo, h_hi, elr)
    h2, elr2 = _mid(olo[0], olo[1], ohi[0], ohi[1], slo[0], slo[1],
                    shi[0], shi[1], E4, b1f[:, :_DP], b1f[:, _DP:],
                    W2p[:_DP], W2p[_DP:], ALR2, _DP)
    o2, s2 = _edge_pass(h2, elr2, src, dst, hoff0)
    return _fin(o2[0], o2[1], s2[0], s2[1], b2f)


# speedup vs baseline: 49.3515x; 1.7405x over previous
import functools

import jax
import jax.numpy as jnp
from jax import lax
from jax.experimental import pallas as pl
from jax.experimental.pallas import tpu as pltpu
from jax.experimental.pallas import tpu_sc as plsc

_N = 10000
_E = 320000
_H = 8
_DH = 16
_NCLS = 40
_DP = 64

_NCORES = 2
_NSUB = 16
_NW = _NCORES * _NSUB
_CH = 128
_NROWS = _E // _CH
_RPW = _NROWS // _NW
_RXTRA = _NROWS - _RPW * _NW
_ZR = 200
_NZB = _N // _ZR


def _first_body(x_ref, w_ref, alr_ref, hlo_ref, hhi_ref, elr_ref):
    h = jnp.dot(x_ref[...], w_ref[...], preferred_element_type=jnp.float32)
    hlo_ref[...] = h[:, :_DP]
    hhi_ref[...] = h[:, _DP:]
    elr_ref[...] = jnp.dot(h, alr_ref[...], preferred_element_type=jnp.float32)


def _make_mid_body(two_halves):
    def _mid_body(olo0_ref, olo1_ref, ohi0_ref, ohi1_ref, slo0_ref, slo1_ref,
                  shi0_ref, shi1_ref, e4_ref, blo_ref, bhi_ref, wlo_ref,
                  whi_ref, alr_ref, *out_refs):
        s_lo = (slo0_ref[...] + slo1_ref[...])[:, :4]
        s_hi = (shi0_ref[...] + shi1_ref[...])[:, :4]
        e4 = e4_ref[...]
        d_lo = jnp.dot(s_lo, e4, preferred_element_type=jnp.float32) + 1e-9
        d_hi = jnp.dot(s_hi, e4, preferred_element_type=jnp.float32) + 1e-9
        x_lo = (olo0_ref[...] + olo1_ref[...]) / d_lo + blo_ref[...]
        x_hi = (ohi0_ref[...] + ohi1_ref[...]) / d_hi + bhi_ref[...]
        x_lo = jnp.maximum(x_lo, 0.0)
        x_hi = jnp.maximum(x_hi, 0.0)
        h = (jnp.dot(x_lo, wlo_ref[...], preferred_element_type=jnp.float32)
             + jnp.dot(x_hi, whi_ref[...], preferred_element_type=jnp.float32))
        if two_halves:
            hlo_ref, hhi_ref, elr_ref = out_refs
            hlo_ref[...] = h[:, :_DP]
            hhi_ref[...] = h[:, _DP:]
        else:
            hlo_ref, elr_ref = out_refs
            hlo_ref[...] = h
        elr_ref[...] = jnp.dot(h, alr_ref[...],
                               preferred_element_type=jnp.float32)
    return _mid_body


def _fin_body(o0_ref, o1_ref, s0_ref, s1_ref, b_ref, out_ref):
    o = o0_ref[...] + o1_ref[...]
    s = (s0_ref[...] + s1_ref[...])[:, 0:1]
    out_ref[...] = o[:, :_NCLS] / (s + 1e-9) + b_ref[...]


_BM = 400


def _first(x, W, ALR):
    return pl.pallas_call(
        _first_body,
        grid=(_N // _BM,),
        in_specs=[pl.BlockSpec((_BM, 128), lambda i: (i, 0)),
                  pl.BlockSpec((128, 128), lambda i: (0, 0)),
                  pl.BlockSpec((128, 16), lambda i: (0, 0))],
        out_specs=[pl.BlockSpec((_BM, _DP), lambda i: (i, 0)),
                   pl.BlockSpec((_BM, _DP), lambda i: (i, 0)),
                   pl.BlockSpec((_BM, 16), lambda i: (i, 0))],
        out_shape=[jax.ShapeDtypeStruct((_N, _DP), jnp.float32),
                   jax.ShapeDtypeStruct((_N, _DP), jnp.float32),
                   jax.ShapeDtypeStruct((_N, 16), jnp.float32)],
    )(x, W, ALR)


def _mid(olo0, olo1, ohi0, ohi1, slo0, slo1, shi0, shi1, E4, b_lo, b_hi,
         W_lo, W_hi, ALR, dp_out):
    two = dp_out == 128
    out_specs = [pl.BlockSpec((_BM, _DP), lambda i: (i, 0))]
    out_shape = [jax.ShapeDtypeStruct((_N, _DP), jnp.float32)]
    if two:
        out_specs.append(pl.BlockSpec((_BM, _DP), lambda i: (i, 0)))
        out_shape.append(jax.ShapeDtypeStruct((_N, _DP), jnp.float32))
    out_specs.append(pl.BlockSpec((_BM, 16), lambda i: (i, 0)))
    out_shape.append(jax.ShapeDtypeStruct((_N, 16), jnp.float32))
    return pl.pallas_call(
        _make_mid_body(two),
        grid=(_N // _BM,),
        in_specs=[pl.BlockSpec((_BM, _DP), lambda i: (i, 0)),
                  pl.BlockSpec((_BM, _DP), lambda i: (i, 0)),
                  pl.BlockSpec((_BM, _DP), lambda i: (i, 0)),
                  pl.BlockSpec((_BM, _DP), lambda i: (i, 0)),
                  pl.BlockSpec((_BM, 16), lambda i: (i, 0)),
                  pl.BlockSpec((_BM, 16), lambda i: (i, 0)),
                  pl.BlockSpec((_BM, 16), lambda i: (i, 0)),
                  pl.BlockSpec((_BM, 16), lambda i: (i, 0)),
                  pl.BlockSpec((4, _DP), lambda i: (0, 0)),
                  pl.BlockSpec((1, _DP), lambda i: (0, 0)),
                  pl.BlockSpec((1, _DP), lambda i: (0, 0)),
                  pl.BlockSpec((_DP, dp_out), lambda i: (0, 0)),
                  pl.BlockSpec((_DP, dp_out), lambda i: (0, 0)),
                  pl.BlockSpec((dp_out, 16), lambda i: (0, 0))],
        out_specs=out_specs,
        out_shape=out_shape,
    )(olo0, olo1, ohi0, ohi1, slo0, slo1, shi0, shi1, E4, b_lo, b_hi,
      W_lo, W_hi, ALR)


def _fin(o0, o1, s0, s1, b):
    return pl.pallas_call(
        _fin_body,
        grid=(_N // _BM,),
        in_specs=[pl.BlockSpec((_BM, _DP), lambda i: (i, 0)),
                  pl.BlockSpec((_BM, _DP), lambda i: (i, 0)),
                  pl.BlockSpec((_BM, 16), lambda i: (i, 0)),
                  pl.BlockSpec((_BM, 16), lambda i: (i, 0)),
                  pl.BlockSpec((1, _NCLS), lambda i: (0, 0))],
        out_specs=pl.BlockSpec((_BM, _NCLS), lambda i: (i, 0)),
        out_shape=jax.ShapeDtypeStruct((_N, _NCLS), jnp.float32),
    )(o0, o1, s0, s1, b)


_mesh = plsc.VectorSubcoreMesh(core_axis_name="c", subcore_axis_name="s")


@functools.partial(
    pl.kernel,
    mesh=_mesh,
    compiler_params=pltpu.CompilerParams(needs_layout_passes=False,
                                         use_tc_tiling_on_sc=False),
    out_type=[jax.ShapeDtypeStruct((_NCORES, _N, _DP), jnp.float32),
              jax.ShapeDtypeStruct((_NCORES, _N, 16), jnp.float32)],
    scratch_types=[
        pltpu.VMEM((16,), jnp.int32),
        pltpu.VMEM((_RPW + 1, _CH), jnp.int32),
        pltpu.VMEM((_RPW + 1, _CH), jnp.int32),
        pltpu.VMEM((2, _CH, 16), jnp.float32),
        pltpu.VMEM((2, _CH, 16), jnp.float32),
        pltpu.VMEM((_CH, 16), jnp.float32),
        pltpu.VMEM((2, _CH, _DP), jnp.float32),
        pltpu.VMEM((_ZR, _DP), jnp.float32),
        pltpu.VMEM((_ZR, 16), jnp.float32),
        pltpu.VMEM_SHARED((_N, _DP), jnp.float32),
        pltpu.VMEM_SHARED((_N, 16), jnp.float32),
        pltpu.SemaphoreType.DMA,
        pltpu.SemaphoreType.DMA,
        pltpu.SemaphoreType.DMA,
        pltpu.SemaphoreType.DMA,
    ],
)
def _edge_pass(h_hbm, elr_hbm, src_hbm, dst_hbm, hoff_hbm, o_out, s_out,
               hoffb, sidx, didx, elrs, elrd, eebuf, hbuf, zbuf, zbufs,
               o_acc, s_acc, semh0, semh1, seme0, seme1):
    cid = lax.axis_index("c")
    sid = lax.axis_index("s")
    wid = sid * _NCORES + cid
    zero16 = jnp.zeros((16,), jnp.float32)
    semh = (semh0, semh1)
    seme = (seme0, seme1)

    pltpu.sync_copy(hoff_hbm, hoffb)
    hv = hoffb[:]

    base = wid * _RPW + jnp.minimum(wid, _RXTRA)
    has_extra = wid < _RXTRA
    pltpu.sync_copy(src_hbm.at[pl.ds(base, _RPW)], sidx.at[pl.ds(0, _RPW)])
    pltpu.sync_copy(dst_hbm.at[pl.ds(base, _RPW)], didx.at[pl.ds(0, _RPW)])
    @pl.when(has_extra)
    def _():
        pltpu.sync_copy(src_hbm.at[pl.ds(base + _RPW, 1)],
                        sidx.at[pl.ds(_RPW, 1)])
        pltpu.sync_copy(dst_hbm.at[pl.ds(base + _RPW, 1)],
                        didx.at[pl.ds(_RPW, 1)])

    def zrow(i, carry):
        for v in range(_DP // 16):
            zbuf[i, pl.ds(16 * v, 16)] = zero16
        zbufs[i, :] = zero16
        return carry
    lax.fori_loop(0, _ZR, zrow, 0)

    def zee(i, carry):
        eebuf[i, :] = zero16
        return carry
    lax.fori_loop(0, _CH, zee, 0)

    for k in range((_NZB + _NSUB - 1) // _NSUB):
        blk = sid + _NSUB * k
        @pl.when(blk < _NZB)
        def _():
            r0 = blk * _ZR
            pltpu.sync_copy(zbuf, o_acc.at[pl.ds(r0, _ZR)])
            pltpu.sync_copy(zbufs, s_acc.at[pl.ds(r0, _ZR)])
    plsc.subcore_barrier()

    iot = lax.iota(jnp.int32, 16)

    def issue(j, p):
        pltpu.async_copy(h_hbm.at[sidx.at[j]], hbuf.at[p], semh[p])
        pltpu.async_copy(elr_hbm.at[sidx.at[j]], elrs.at[p], seme[p])
        pltpu.async_copy(elr_hbm.at[didx.at[j]], elrd.at[p], seme[p])

    def wait(j, p):
        pltpu.make_async_copy(h_hbm.at[sidx.at[j]], hbuf.at[p],
                              semh[p]).wait()
        pltpu.make_async_copy(elr_hbm.at[sidx.at[j]], elrs.at[p],
                              seme[p]).wait()
        pltpu.make_async_copy(elr_hbm.at[didx.at[j]], elrd.at[p],
                              seme[p]).wait()

    def process(j, p):
        hb = hbuf.at[p]
        es = elrs.at[p]
        ed = elrd.at[p]
        for t in range(_CH // 16):
            rows = iot + (16 * t)
            for jj in range(4):
                el16 = plsc.load_gather(es, [rows, hv + jj])
                er16 = plsc.load_gather(ed, [rows, hv + (8 + jj)])
                e = el16 + er16
                e = jnp.where(e >= 0.0, e, 0.2 * e)
                plsc.store_scatter(
                    eebuf, [rows, jnp.full((16,), jj, jnp.int32)], jnp.exp(e))
        pltpu.sync_copy(eebuf, s_acc.at[didx.at[j]], add=True)

        def medge(i, c2):
            iv = jnp.full((16,), i, jnp.int32)
            for v in range(_DP // 16):
                w = plsc.load_gather(eebuf,
                                     [iv, jnp.full((16,), v, jnp.int32)])
                hval = hb[i, pl.ds(16 * v, 16)]
                hb[i, pl.ds(16 * v, 16)] = hval * w
            return c2
        lax.fori_loop(0, _CH, medge, 0)
        pltpu.sync_copy(hb, o_acc.at[didx.at[j]], add=True)

    nrows = _RPW + has_extra.astype(jnp.int32)
    issue(0, 0)

    def pair_body(pr, carry):
        j0 = 2 * pr
        issue(j0 + 1, 1)
        wait(j0, 0)
        process(j0, 0)
        @pl.when(j0 + 2 < nrows)
        def _():
            issue(j0 + 2, 0)
        wait(j0 + 1, 1)
        process(j0 + 1, 1)
        return carry
    lax.fori_loop(0, _RPW // 2, pair_body, 0)

    @pl.when(has_extra)
    def _():
        wait(_RPW, 0)
        process(_RPW, 0)

    plsc.subcore_barrier()
    for k in range((_NZB + _NSUB - 1) // _NSUB):
        blk = sid + _NSUB * k
        @pl.when(blk < _NZB)
        def _():
            r0 = blk * _ZR
            pltpu.sync_copy(o_acc.at[pl.ds(r0, _ZR)],
                            o_out.at[cid, pl.ds(r0, _ZR)])
            pltpu.sync_copy(s_acc.at[pl.ds(r0, _ZR)],
                            s_out.at[cid, pl.ds(r0, _ZR)])


def _alr_mat(al, ar, heads, dout):
    alf = al.reshape(-1)
    arf = ar.reshape(-1)
    d = heads * dout
    mask = (jnp.arange(d)[:, None] // dout == jnp.arange(heads)[None, :])
    mask = mask.astype(jnp.float32)
    ALm = mask * alf[:, None]
    ARm = mask * arf[:, None]
    pad = jnp.zeros((d, 8 - heads), jnp.float32)
    return jnp.concatenate([ALm, pad, ARm, pad], axis=1)


def kernel(features, edge_index, W0, al0, ar0, b0, W1, al1, ar1, b1,
           W2, al2, ar2, b2):
    src = edge_index[0].reshape(_NROWS, _CH)
    dst = edge_index[1].reshape(_NROWS, _CH)

    ALR0 = _alr_mat(al0, ar0, _H, _DH)
    ALR1 = _alr_mat(al1, ar1, _H, _DH)
    al2f = jnp.pad(al2.reshape(-1), (0, _DP - _NCLS))
    ar2f = jnp.pad(ar2.reshape(-1), (0, _DP - _NCLS))
    rep = (jnp.arange(16)[None, :] < 4).astype(jnp.float32)
    rep_r = ((jnp.arange(16)[None, :] >= 8)
             & (jnp.arange(16)[None, :] < 12)).astype(jnp.float32)
    ALR2 = al2f[:, None] * rep + ar2f[:, None] * rep_r
    W2p = jnp.pad(W2, ((0, 0), (0, _DP - _NCLS)))
    E4 = (jnp.arange(_DP)[None, :] // 16 == jnp.arange(4)[:, None])
    E4 = E4.astype(jnp.float32)
    b0f = b0.reshape(1, 128)
    b1f = b1.reshape(1, 128)
    b2f = b2.reshape(1, _NCLS)
    hoff0 = jnp.zeros((16,), jnp.int32)
    hoff4 = jnp.full((16,), 4, jnp.int32)

    def run_layer(h_lo, h_hi, elr):
        olo, slo = _edge_pass(h_lo, elr, src, dst, hoff0)
        ohi, shi = _edge_pass(h_hi, elr, src, dst, hoff4)
        return olo, ohi, slo, shi

    h_lo, h_hi, elr = _first(features, W0, ALR0)
    olo, ohi, slo, shi = run_layer(h_lo, h_hi, elr)
    h_lo, h_hi, elr = _mid(olo[0], olo[1], ohi[0], ohi[1], slo[0], slo[1],
                           shi[0], shi[1], E4, b0f[:, :_DP], b0f[:, _DP:],
                           W1[:_DP], W1[_DP:], ALR1, 128)
    olo, ohi, slo, shi = run_layer(h_lo, h_hi, elr)
    h2, elr2 = _mid(olo[0], olo[1], ohi[0], ohi[1], slo[0], slo[1],
                    shi[0], shi[1], E4, b1f[:, :_DP], b1f[:, _DP:],
                    W2p[:_DP], W2p[_DP:], ALR2, _DP)
    o2, s2 = _edge_pass(h2, elr2, src, dst, hoff0)
    return _fin(o2[0], o2[1], s2[0], s2[1], b2f)


# speedup vs baseline: 90.2634x; 1.8290x over previous
import functools

import jax
import jax.numpy as jnp
from jax import lax
from jax.experimental import pallas as pl
from jax.experimental.pallas import tpu as pltpu
from jax.experimental.pallas import tpu_sc as plsc

_N = 10000
_E = 320000
_H = 8
_DH = 16
_NCLS = 40
_DP = 64

_NCORES = 2
_NSUB = 16
_NW = _NCORES * _NSUB
_CH = 128
_NROWS = _E // _CH
_RPW = _NROWS // _NW
_RXTRA = _NROWS - _RPW * _NW
_ZR = 200
_NZB = _N // _ZR


def _first_body(x_ref, w_ref, alr_ref, hlo_ref, hhi_ref, elr_ref):
    h = jnp.dot(x_ref[...], w_ref[...], preferred_element_type=jnp.float32)
    hlo_ref[...] = h[:, :_DP]
    hhi_ref[...] = h[:, _DP:]
    elr_ref[...] = jnp.dot(h, alr_ref[...], preferred_element_type=jnp.float32)


def _make_mid_body(two_halves):
    def _mid_body(olo0_ref, olo1_ref, ohi0_ref, ohi1_ref, slo0_ref, slo1_ref,
                  shi0_ref, shi1_ref, e4_ref, blo_ref, bhi_ref, wlo_ref,
                  whi_ref, alr_ref, *out_refs):
        s_lo = (slo0_ref[...] + slo1_ref[...])[:, :4]
        s_hi = (shi0_ref[...] + shi1_ref[...])[:, :4]
        e4 = e4_ref[...]
        d_lo = jnp.dot(s_lo, e4, preferred_element_type=jnp.float32) + 1e-9
        d_hi = jnp.dot(s_hi, e4, preferred_element_type=jnp.float32) + 1e-9
        x_lo = (olo0_ref[...] + olo1_ref[...]) / d_lo + blo_ref[...]
        x_hi = (ohi0_ref[...] + ohi1_ref[...]) / d_hi + bhi_ref[...]
        x_lo = jnp.maximum(x_lo, 0.0)
        x_hi = jnp.maximum(x_hi, 0.0)
        h = (jnp.dot(x_lo, wlo_ref[...], preferred_element_type=jnp.float32)
             + jnp.dot(x_hi, whi_ref[...], preferred_element_type=jnp.float32))
        if two_halves:
            hlo_ref, hhi_ref, elr_ref = out_refs
            hlo_ref[...] = h[:, :_DP]
            hhi_ref[...] = h[:, _DP:]
        else:
            hlo_ref, elr_ref = out_refs
            hlo_ref[...] = h
        elr_ref[...] = jnp.dot(h, alr_ref[...],
                               preferred_element_type=jnp.float32)
    return _mid_body


def _fin_body(o0_ref, o1_ref, s0_ref, s1_ref, b_ref, out_ref):
    o = o0_ref[...] + o1_ref[...]
    s = (s0_ref[...] + s1_ref[...])[:, 0:1]
    out_ref[...] = o[:, :_NCLS] / (s + 1e-9) + b_ref[...]


_BM = 400


def _first(x, W, ALR):
    return pl.pallas_call(
        _first_body,
        grid=(_N // _BM,),
        in_specs=[pl.BlockSpec((_BM, 128), lambda i: (i, 0)),
                  pl.BlockSpec((128, 128), lambda i: (0, 0)),
                  pl.BlockSpec((128, 16), lambda i: (0, 0))],
        out_specs=[pl.BlockSpec((_BM, _DP), lambda i: (i, 0)),
                   pl.BlockSpec((_BM, _DP), lambda i: (i, 0)),
                   pl.BlockSpec((_BM, 16), lambda i: (i, 0))],
        out_shape=[jax.ShapeDtypeStruct((_N, _DP), jnp.float32),
                   jax.ShapeDtypeStruct((_N, _DP), jnp.float32),
                   jax.ShapeDtypeStruct((_N, 16), jnp.float32)],
    )(x, W, ALR)


def _mid(olo0, olo1, ohi0, ohi1, slo0, slo1, shi0, shi1, E4, b_lo, b_hi,
         W_lo, W_hi, ALR, dp_out):
    two = dp_out == 128
    out_specs = [pl.BlockSpec((_BM, _DP), lambda i: (i, 0))]
    out_shape = [jax.ShapeDtypeStruct((_N, _DP), jnp.float32)]
    if two:
        out_specs.append(pl.BlockSpec((_BM, _DP), lambda i: (i, 0)))
        out_shape.append(jax.ShapeDtypeStruct((_N, _DP), jnp.float32))
    out_specs.append(pl.BlockSpec((_BM, 16), lambda i: (i, 0)))
    out_shape.append(jax.ShapeDtypeStruct((_N, 16), jnp.float32))
    return pl.pallas_call(
        _make_mid_body(two),
        grid=(_N // _BM,),
        in_specs=[pl.BlockSpec((_BM, _DP), lambda i: (i, 0)),
                  pl.BlockSpec((_BM, _DP), lambda i: (i, 0)),
                  pl.BlockSpec((_BM, _DP), lambda i: (i, 0)),
                  pl.BlockSpec((_BM, _DP), lambda i: (i, 0)),
                  pl.BlockSpec((_BM, 16), lambda i: (i, 0)),
                  pl.BlockSpec((_BM, 16), lambda i: (i, 0)),
                  pl.BlockSpec((_BM, 16), lambda i: (i, 0)),
                  pl.BlockSpec((_BM, 16), lambda i: (i, 0)),
                  pl.BlockSpec((4, _DP), lambda i: (0, 0)),
                  pl.BlockSpec((1, _DP), lambda i: (0, 0)),
                  pl.BlockSpec((1, _DP), lambda i: (0, 0)),
                  pl.BlockSpec((_DP, dp_out), lambda i: (0, 0)),
                  pl.BlockSpec((_DP, dp_out), lambda i: (0, 0)),
                  pl.BlockSpec((dp_out, 16), lambda i: (0, 0))],
        out_specs=out_specs,
        out_shape=out_shape,
    )(olo0, olo1, ohi0, ohi1, slo0, slo1, shi0, shi1, E4, b_lo, b_hi,
      W_lo, W_hi, ALR)


def _fin(o0, o1, s0, s1, b):
    return pl.pallas_call(
        _fin_body,
        grid=(_N // _BM,),
        in_specs=[pl.BlockSpec((_BM, _DP), lambda i: (i, 0)),
                  pl.BlockSpec((_BM, _DP), lambda i: (i, 0)),
                  pl.BlockSpec((_BM, 16), lambda i: (i, 0)),
                  pl.BlockSpec((_BM, 16), lambda i: (i, 0)),
                  pl.BlockSpec((1, _NCLS), lambda i: (0, 0))],
        out_specs=pl.BlockSpec((_BM, _NCLS), lambda i: (i, 0)),
        out_shape=jax.ShapeDtypeStruct((_N, _NCLS), jnp.float32),
    )(o0, o1, s0, s1, b)


_mesh = plsc.VectorSubcoreMesh(core_axis_name="c", subcore_axis_name="s")


@functools.partial(
    pl.kernel,
    mesh=_mesh,
    compiler_params=pltpu.CompilerParams(needs_layout_passes=False,
                                         use_tc_tiling_on_sc=False),
    out_type=[jax.ShapeDtypeStruct((_NCORES, _N, _DP), jnp.float32),
              jax.ShapeDtypeStruct((_NCORES, _N, 16), jnp.float32)],
    scratch_types=[
        pltpu.VMEM((16,), jnp.int32),
        pltpu.VMEM((_RPW + 1, _CH), jnp.int32),
        pltpu.VMEM((_RPW + 1, _CH), jnp.int32),
        pltpu.VMEM((2, _CH, 16), jnp.float32),
        pltpu.VMEM((2, _CH, 16), jnp.float32),
        pltpu.VMEM((_CH, 16), jnp.float32),
        pltpu.VMEM((2, _CH, _DP), jnp.float32),
        pltpu.VMEM((_ZR, _DP), jnp.float32),
        pltpu.VMEM((_ZR, 16), jnp.float32),
        pltpu.VMEM_SHARED((_N, _DP), jnp.float32),
        pltpu.VMEM_SHARED((_N, 16), jnp.float32),
        pltpu.SemaphoreType.DMA,
        pltpu.SemaphoreType.DMA,
        pltpu.SemaphoreType.DMA,
        pltpu.SemaphoreType.DMA,
    ],
)
def _edge_pass(h_hbm, elr_hbm, src_hbm, dst_hbm, hoff_hbm, o_out, s_out,
               hoffb, sidx, didx, elrs, elrd, eebuf, hbuf, zbuf, zbufs,
               o_acc, s_acc, semh0, semh1, seme0, seme1):
    cid = lax.axis_index("c")
    sid = lax.axis_index("s")
    wid = sid * _NCORES + cid
    zero16 = jnp.zeros((16,), jnp.float32)
    semh = (semh0, semh1)
    seme = (seme0, seme1)

    pltpu.sync_copy(hoff_hbm, hoffb)
    hv = hoffb[:]

    base = wid * _RPW + jnp.minimum(wid, _RXTRA)
    has_extra = wid < _RXTRA
    pltpu.sync_copy(src_hbm.at[pl.ds(base, _RPW)], sidx.at[pl.ds(0, _RPW)])
    pltpu.sync_copy(dst_hbm.at[pl.ds(base, _RPW)], didx.at[pl.ds(0, _RPW)])
    @pl.when(has_extra)
    def _():
        pltpu.sync_copy(src_hbm.at[pl.ds(base + _RPW, 1)],
                        sidx.at[pl.ds(_RPW, 1)])
        pltpu.sync_copy(dst_hbm.at[pl.ds(base + _RPW, 1)],
                        didx.at[pl.ds(_RPW, 1)])

    def zrow(i, carry):
        for v in range(_DP // 16):
            zbuf[i, pl.ds(16 * v, 16)] = zero16
        zbufs[i, :] = zero16
        return carry
    lax.fori_loop(0, _ZR, zrow, 0)

    def zee(i, carry):
        eebuf[i, :] = zero16
        return carry
    lax.fori_loop(0, _CH, zee, 0)

    for k in range((_NZB + _NSUB - 1) // _NSUB):
        blk = sid + _NSUB * k
        @pl.when(blk < _NZB)
        def _():
            r0 = blk * _ZR
            pltpu.sync_copy(zbuf, o_acc.at[pl.ds(r0, _ZR)])
            pltpu.sync_copy(zbufs, s_acc.at[pl.ds(r0, _ZR)])
    plsc.subcore_barrier()

    iot = lax.iota(jnp.int32, 16)

    def issue(j, p):
        pltpu.async_copy(h_hbm.at[sidx.at[j]], hbuf.at[p], semh[p])
        pltpu.async_copy(elr_hbm.at[sidx.at[j]], elrs.at[p], seme[p])
        pltpu.async_copy(elr_hbm.at[didx.at[j]], elrd.at[p], seme[p])

    def wait(j, p):
        pltpu.make_async_copy(h_hbm.at[sidx.at[j]], hbuf.at[p],
                              semh[p]).wait()
        pltpu.make_async_copy(elr_hbm.at[sidx.at[j]], elrs.at[p],
                              seme[p]).wait()
        pltpu.make_async_copy(elr_hbm.at[didx.at[j]], elrd.at[p],
                              seme[p]).wait()

    def process(j, p):
        hb = hbuf.at[p]
        es = elrs.at[p]
        ed = elrd.at[p]
        for t in range(_CH // 16):
            rows = iot + (16 * t)
            for jj in range(4):
                el16 = plsc.load_gather(es, [rows, hv + jj])
                er16 = plsc.load_gather(ed, [rows, hv + (8 + jj)])
                e = el16 + er16
                e = jnp.where(e >= 0.0, e, 0.2 * e)
                plsc.store_scatter(
                    eebuf, [rows, jnp.full((16,), jj, jnp.int32)], jnp.exp(e))
        pltpu.sync_copy(eebuf, s_acc.at[didx.at[j]], add=True)

        @plsc.parallel_loop(0, _CH, unroll=4)
        def medge(i):
            eerow = eebuf[i, :]
            for v in range(_DP // 16):
                hval = hb[i, pl.ds(16 * v, 16)]
                hb[i, pl.ds(16 * v, 16)] = hval * eerow[v]
        pltpu.sync_copy(hb, o_acc.at[didx.at[j]], add=True)

    nrows = _RPW + has_extra.astype(jnp.int32)
    issue(0, 0)

    def pair_body(pr, carry):
        j0 = 2 * pr
        issue(j0 + 1, 1)
        wait(j0, 0)
        process(j0, 0)
        @pl.when(j0 + 2 < nrows)
        def _():
            issue(j0 + 2, 0)
        wait(j0 + 1, 1)
        process(j0 + 1, 1)
        return carry
    lax.fori_loop(0, _RPW // 2, pair_body, 0)

    @pl.when(has_extra)
    def _():
        wait(_RPW, 0)
        process(_RPW, 0)

    plsc.subcore_barrier()
    for k in range((_NZB + _NSUB - 1) // _NSUB):
        blk = sid + _NSUB * k
        @pl.when(blk < _NZB)
        def _():
            r0 = blk * _ZR
            pltpu.sync_copy(o_acc.at[pl.ds(r0, _ZR)],
                            o_out.at[cid, pl.ds(r0, _ZR)])
            pltpu.sync_copy(s_acc.at[pl.ds(r0, _ZR)],
                            s_out.at[cid, pl.ds(r0, _ZR)])


def _alr_mat(al, ar, heads, dout):
    alf = al.reshape(-1)
    arf = ar.reshape(-1)
    d = heads * dout
    mask = (jnp.arange(d)[:, None] // dout == jnp.arange(heads)[None, :])
    mask = mask.astype(jnp.float32)
    ALm = mask * alf[:, None]
    ARm = mask * arf[:, None]
    pad = jnp.zeros((d, 8 - heads), jnp.float32)
    return jnp.concatenate([ALm, pad, ARm, pad], axis=1)


def kernel(features, edge_index, W0, al0, ar0, b0, W1, al1, ar1, b1,
           W2, al2, ar2, b2):
    src = edge_index[0].reshape(_NROWS, _CH)
    dst = edge_index[1].reshape(_NROWS, _CH)

    ALR0 = _alr_mat(al0, ar0, _H, _DH)
    ALR1 = _alr_mat(al1, ar1, _H, _DH)
    al2f = jnp.pad(al2.reshape(-1), (0, _DP - _NCLS))
    ar2f = jnp.pad(ar2.reshape(-1), (0, _DP - _NCLS))
    rep = (jnp.arange(16)[None, :] < 4).astype(jnp.float32)
    rep_r = ((jnp.arange(16)[None, :] >= 8)
             & (jnp.arange(16)[None, :] < 12)).astype(jnp.float32)
    ALR2 = al2f[:, None] * rep + ar2f[:, None] * rep_r
    W2p = jnp.pad(W2, ((0, 0), (0, _DP - _NCLS)))
    E4 = (jnp.arange(_DP)[None, :] // 16 == jnp.arange(4)[:, None])
    E4 = E4.astype(jnp.float32)
    b0f = b0.reshape(1, 128)
    b1f = b1.reshape(1, 128)
    b2f = b2.reshape(1, _NCLS)
    hoff0 = jnp.zeros((16,), jnp.int32)
    hoff4 = jnp.full((16,), 4, jnp.int32)

    def run_layer(h_lo, h_hi, elr):
        olo, slo = _edge_pass(h_lo, elr, src, dst, hoff0)
        ohi, shi = _edge_pass(h_hi, elr, src, dst, hoff4)
        return olo, ohi, slo, shi

    h_lo, h_hi, elr = _first(features, W0, ALR0)
    olo, ohi, slo, shi = run_layer(h_lo, h_hi, elr)
    h_lo, h_hi, elr = _mid(olo[0], olo[1], ohi[0], ohi[1], slo[0], slo[1],
                           shi[0], shi[1], E4, b0f[:, :_DP], b0f[:, _DP:],
                           W1[:_DP], W1[_DP:], ALR1, 128)
    olo, ohi, slo, shi = run_layer(h_lo, h_hi, elr)
    h2, elr2 = _mid(olo[0], olo[1], ohi[0], ohi[1], slo[0], slo[1],
                    shi[0], shi[1], E4, b1f[:, :_DP], b1f[:, _DP:],
                    W2p[:_DP], W2p[_DP:], ALR2, _DP)
    o2, s2 = _edge_pass(h2, elr2, src, dst, hoff0)
    return _fin(o2[0], o2[1], s2[0], s2[1], b2f)


# speedup vs baseline: 114.2648x; 1.2659x over previous
import functools

import jax
import jax.numpy as jnp
from jax import lax
from jax.experimental import pallas as pl
from jax.experimental.pallas import tpu as pltpu
from jax.experimental.pallas import tpu_sc as plsc

_N = 10000
_E = 320000
_H = 8
_DH = 16
_NCLS = 40
_DP = 64

_NCORES = 2
_NSUB = 16
_NW = _NCORES * _NSUB
_CH = 128
_NROWS = _E // _CH
_RPW = _NROWS // _NW
_RXTRA = _NROWS - _RPW * _NW
_ZR = 200
_NZB = _N // _ZR


def _first_body(x_ref, w_ref, alr_ref, hlo_ref, hhi_ref, ello_ref, erlo_ref,
                elhi_ref, erhi_ref):
    h = jnp.dot(x_ref[...], w_ref[...], preferred_element_type=jnp.float32)
    hlo_ref[...] = h[:, :_DP]
    hhi_ref[...] = h[:, _DP:]
    t = jnp.dot(h, alr_ref[...], preferred_element_type=jnp.float32)
    ello_ref[...] = t[:, 0:16]
    erlo_ref[...] = t[:, 16:32]
    elhi_ref[...] = t[:, 32:48]
    erhi_ref[...] = t[:, 48:64]


def _make_mid_body(two_halves):
    def _mid_body(olo0_ref, olo1_ref, ohi0_ref, ohi1_ref, slo0_ref, slo1_ref,
                  shi0_ref, shi1_ref, e4_ref, blo_ref, bhi_ref, wlo_ref,
                  whi_ref, alr_ref, *out_refs):
        s_lo = (slo0_ref[...] + slo1_ref[...])[:, :4]
        s_hi = (shi0_ref[...] + shi1_ref[...])[:, :4]
        e4 = e4_ref[...]
        d_lo = jnp.dot(s_lo, e4, preferred_element_type=jnp.float32) + 1e-9
        d_hi = jnp.dot(s_hi, e4, preferred_element_type=jnp.float32) + 1e-9
        x_lo = (olo0_ref[...] + olo1_ref[...]) / d_lo + blo_ref[...]
        x_hi = (ohi0_ref[...] + ohi1_ref[...]) / d_hi + bhi_ref[...]
        x_lo = jnp.maximum(x_lo, 0.0)
        x_hi = jnp.maximum(x_hi, 0.0)
        h = (jnp.dot(x_lo, wlo_ref[...], preferred_element_type=jnp.float32)
             + jnp.dot(x_hi, whi_ref[...], preferred_element_type=jnp.float32))
        t = jnp.dot(h, alr_ref[...], preferred_element_type=jnp.float32)
        if two_halves:
            hlo_ref, hhi_ref, ello_ref, erlo_ref, elhi_ref, erhi_ref = out_refs
            hlo_ref[...] = h[:, :_DP]
            hhi_ref[...] = h[:, _DP:]
            elhi_ref[...] = t[:, 32:48]
            erhi_ref[...] = t[:, 48:64]
        else:
            hlo_ref, ello_ref, erlo_ref = out_refs
            hlo_ref[...] = h
        ello_ref[...] = t[:, 0:16]
        erlo_ref[...] = t[:, 16:32]
    return _mid_body


def _fin_body(o0_ref, o1_ref, s0_ref, s1_ref, b_ref, out_ref):
    o = o0_ref[...] + o1_ref[...]
    s = (s0_ref[...] + s1_ref[...])[:, 0:1]
    out_ref[...] = o[:, :_NCLS] / (s + 1e-9) + b_ref[...]


_BM = 400


_SPEC16 = [pl.BlockSpec((_BM, 16), lambda i: (i, 0)) for _ in range(4)]
_SHAPE16 = [jax.ShapeDtypeStruct((_N, 16), jnp.float32) for _ in range(4)]


def _first(x, W, ALR):
    return pl.pallas_call(
        _first_body,
        grid=(_N // _BM,),
        in_specs=[pl.BlockSpec((_BM, 128), lambda i: (i, 0)),
                  pl.BlockSpec((128, 128), lambda i: (0, 0)),
                  pl.BlockSpec((128, 64), lambda i: (0, 0))],
        out_specs=[pl.BlockSpec((_BM, _DP), lambda i: (i, 0)),
                   pl.BlockSpec((_BM, _DP), lambda i: (i, 0))] + _SPEC16,
        out_shape=[jax.ShapeDtypeStruct((_N, _DP), jnp.float32),
                   jax.ShapeDtypeStruct((_N, _DP), jnp.float32)] + _SHAPE16,
    )(x, W, ALR)


def _mid(olo0, olo1, ohi0, ohi1, slo0, slo1, shi0, shi1, E4, b_lo, b_hi,
         W_lo, W_hi, ALR, dp_out):
    two = dp_out == 128
    out_specs = [pl.BlockSpec((_BM, _DP), lambda i: (i, 0))]
    out_shape = [jax.ShapeDtypeStruct((_N, _DP), jnp.float32)]
    if two:
        out_specs.append(pl.BlockSpec((_BM, _DP), lambda i: (i, 0)))
        out_shape.append(jax.ShapeDtypeStruct((_N, _DP), jnp.float32))
        out_specs += _SPEC16
        out_shape += _SHAPE16
    else:
        out_specs += _SPEC16[:2]
        out_shape += _SHAPE16[:2]
    return pl.pallas_call(
        _make_mid_body(two),
        grid=(_N // _BM,),
        in_specs=[pl.BlockSpec((_BM, _DP), lambda i: (i, 0)),
                  pl.BlockSpec((_BM, _DP), lambda i: (i, 0)),
                  pl.BlockSpec((_BM, _DP), lambda i: (i, 0)),
                  pl.BlockSpec((_BM, _DP), lambda i: (i, 0)),
                  pl.BlockSpec((_BM, 16), lambda i: (i, 0)),
                  pl.BlockSpec((_BM, 16), lambda i: (i, 0)),
                  pl.BlockSpec((_BM, 16), lambda i: (i, 0)),
                  pl.BlockSpec((_BM, 16), lambda i: (i, 0)),
                  pl.BlockSpec((4, _DP), lambda i: (0, 0)),
                  pl.BlockSpec((1, _DP), lambda i: (0, 0)),
                  pl.BlockSpec((1, _DP), lambda i: (0, 0)),
                  pl.BlockSpec((_DP, dp_out), lambda i: (0, 0)),
                  pl.BlockSpec((_DP, dp_out), lambda i: (0, 0)),
                  pl.BlockSpec((dp_out, 64 if two else 32),
                               lambda i: (0, 0))],
        out_specs=out_specs,
        out_shape=out_shape,
    )(olo0, olo1, ohi0, ohi1, slo0, slo1, shi0, shi1, E4, b_lo, b_hi,
      W_lo, W_hi, ALR)


def _fin(o0, o1, s0, s1, b):
    return pl.pallas_call(
        _fin_body,
        grid=(_N // _BM,),
        in_specs=[pl.BlockSpec((_BM, _DP), lambda i: (i, 0)),
                  pl.BlockSpec((_BM, _DP), lambda i: (i, 0)),
                  pl.BlockSpec((_BM, 16), lambda i: (i, 0)),
                  pl.BlockSpec((_BM, 16), lambda i: (i, 0)),
                  pl.BlockSpec((1, _NCLS), lambda i: (0, 0))],
        out_specs=pl.BlockSpec((_BM, _NCLS), lambda i: (i, 0)),
        out_shape=jax.ShapeDtypeStruct((_N, _NCLS), jnp.float32),
    )(o0, o1, s0, s1, b)


_mesh = plsc.VectorSubcoreMesh(core_axis_name="c", subcore_axis_name="s")


@functools.partial(
    pl.kernel,
    mesh=_mesh,
    compiler_params=pltpu.CompilerParams(needs_layout_passes=False,
                                         use_tc_tiling_on_sc=False),
    out_type=[jax.ShapeDtypeStruct((_NCORES, _N, _DP), jnp.float32),
              jax.ShapeDtypeStruct((_NCORES, _N, 16), jnp.float32)],
    scratch_types=[
        pltpu.VMEM((_RPW + 1, _CH), jnp.int32),
        pltpu.VMEM((_RPW + 1, _CH), jnp.int32),
        pltpu.VMEM((2, _CH, 16), jnp.float32),
        pltpu.VMEM((2, _CH, 16), jnp.float32),
        pltpu.VMEM((_CH, 16), jnp.float32),
        pltpu.VMEM((2, _CH, _DP), jnp.float32),
        pltpu.VMEM((_ZR, _DP), jnp.float32),
        pltpu.VMEM((_ZR, 16), jnp.float32),
        pltpu.VMEM_SHARED((_N, _DP), jnp.float32),
        pltpu.VMEM_SHARED((_N, 16), jnp.float32),
        pltpu.SemaphoreType.DMA,
        pltpu.SemaphoreType.DMA,
        pltpu.SemaphoreType.DMA,
        pltpu.SemaphoreType.DMA,
    ],
)
def _edge_pass(h_hbm, el_hbm, er_hbm, src_hbm, dst_hbm, o_out, s_out,
               sidx, didx, elrs, elrd, eebuf, hbuf, zbuf, zbufs,
               o_acc, s_acc, semh0, semh1, seme0, seme1):
    cid = lax.axis_index("c")
    sid = lax.axis_index("s")
    wid = sid * _NCORES + cid
    zero16 = jnp.zeros((16,), jnp.float32)
    semh = (semh0, semh1)
    seme = (seme0, seme1)

    base = wid * _RPW + jnp.minimum(wid, _RXTRA)
    has_extra = wid < _RXTRA
    pltpu.sync_copy(src_hbm.at[pl.ds(base, _RPW)], sidx.at[pl.ds(0, _RPW)])
    pltpu.sync_copy(dst_hbm.at[pl.ds(base, _RPW)], didx.at[pl.ds(0, _RPW)])
    @pl.when(has_extra)
    def _():
        pltpu.sync_copy(src_hbm.at[pl.ds(base + _RPW, 1)],
                        sidx.at[pl.ds(_RPW, 1)])
        pltpu.sync_copy(dst_hbm.at[pl.ds(base + _RPW, 1)],
                        didx.at[pl.ds(_RPW, 1)])

    def zrow(i, carry):
        for v in range(_DP // 16):
            zbuf[i, pl.ds(16 * v, 16)] = zero16
        zbufs[i, :] = zero16
        return carry
    lax.fori_loop(0, _ZR, zrow, 0)

    for k in range((_NZB + _NSUB - 1) // _NSUB):
        blk = sid + _NSUB * k
        @pl.when(blk < _NZB)
        def _():
            r0 = blk * _ZR
            pltpu.sync_copy(zbuf, o_acc.at[pl.ds(r0, _ZR)])
            pltpu.sync_copy(zbufs, s_acc.at[pl.ds(r0, _ZR)])
    plsc.subcore_barrier()

    def issue(j, p):
        pltpu.async_copy(h_hbm.at[sidx.at[j]], hbuf.at[p], semh[p])
        pltpu.async_copy(el_hbm.at[sidx.at[j]], elrs.at[p], seme[p])
        pltpu.async_copy(er_hbm.at[didx.at[j]], elrd.at[p], seme[p])

    def wait(j, p):
        pltpu.make_async_copy(h_hbm.at[sidx.at[j]], hbuf.at[p],
                              semh[p]).wait()
        pltpu.make_async_copy(el_hbm.at[sidx.at[j]], elrs.at[p],
                              seme[p]).wait()
        pltpu.make_async_copy(er_hbm.at[didx.at[j]], elrd.at[p],
                              seme[p]).wait()

    def process(j, p):
        hb = hbuf.at[p]
        es = elrs.at[p]
        ed = elrd.at[p]

        @plsc.parallel_loop(0, _CH, unroll=4)
        def edge(i):
            e = es[i, :] + ed[i, :]
            e = jnp.where(e >= 0.0, e, 0.2 * e)
            ee = jnp.exp(e)
            eebuf[i, :] = ee
            for v in range(_DP // 16):
                hval = hb[i, pl.ds(16 * v, 16)]
                hb[i, pl.ds(16 * v, 16)] = hval * ee[v]
        pltpu.sync_copy(eebuf, s_acc.at[didx.at[j]], add=True)
        pltpu.sync_copy(hb, o_acc.at[didx.at[j]], add=True)

    nrows = _RPW + has_extra.astype(jnp.int32)
    issue(0, 0)

    def pair_body(pr, carry):
        j0 = 2 * pr
        issue(j0 + 1, 1)
        wait(j0, 0)
        process(j0, 0)
        @pl.when(j0 + 2 < nrows)
        def _():
            issue(j0 + 2, 0)
        wait(j0 + 1, 1)
        process(j0 + 1, 1)
        return carry
    lax.fori_loop(0, _RPW // 2, pair_body, 0)

    @pl.when(has_extra)
    def _():
        wait(_RPW, 0)
        process(_RPW, 0)

    plsc.subcore_barrier()
    for k in range((_NZB + _NSUB - 1) // _NSUB):
        blk = sid + _NSUB * k
        @pl.when(blk < _NZB)
        def _():
            r0 = blk * _ZR
            pltpu.sync_copy(o_acc.at[pl.ds(r0, _ZR)],
                            o_out.at[cid, pl.ds(r0, _ZR)])
            pltpu.sync_copy(s_acc.at[pl.ds(r0, _ZR)],
                            s_out.at[cid, pl.ds(r0, _ZR)])


def _head_tab(vflat, dout, h0, replicate):
    d = vflat.shape[0]
    head = jnp.arange(d)[:, None] // dout
    j = jnp.arange(16)[None, :]
    sel = h0 if replicate else h0 + j
    mask = ((j < 4) & (head == sel)).astype(jnp.float32)
    return mask * vflat[:, None]


def kernel(features, edge_index, W0, al0, ar0, b0, W1, al1, ar1, b1,
           W2, al2, ar2, b2):
    src = edge_index[0].reshape(_NROWS, _CH)
    dst = edge_index[1].reshape(_NROWS, _CH)

    def alr128(al, ar):
        alf = al.reshape(-1)
        arf = ar.reshape(-1)
        return jnp.concatenate(
            [_head_tab(alf, _DH, 0, False), _head_tab(arf, _DH, 0, False),
             _head_tab(alf, _DH, 4, False), _head_tab(arf, _DH, 4, False)],
            axis=1)

    ALR0 = alr128(al0, ar0)
    ALR1 = alr128(al1, ar1)
    al2f = jnp.pad(al2.reshape(-1), (0, _DP - _NCLS))
    ar2f = jnp.pad(ar2.reshape(-1), (0, _DP - _NCLS))
    ALR2 = jnp.concatenate([_head_tab(al2f, _NCLS, 0, True),
                            _head_tab(ar2f, _NCLS, 0, True)], axis=1)
    W2p = jnp.pad(W2, ((0, 0), (0, _DP - _NCLS)))
    E4 = (jnp.arange(_DP)[None, :] // 16 == jnp.arange(4)[:, None])
    E4 = E4.astype(jnp.float32)
    b0f = b0.reshape(1, 128)
    b1f = b1.reshape(1, 128)
    b2f = b2.reshape(1, _NCLS)

    def run_layer(h_lo, h_hi, el_lo, er_lo, el_hi, er_hi):
        olo, slo = _edge_pass(h_lo, el_lo, er_lo, src, dst)
        ohi, shi = _edge_pass(h_hi, el_hi, er_hi, src, dst)
        return olo, ohi, slo, shi

    hl, hh, ell, erl, elh, erh = _first(features, W0, ALR0)
    olo, ohi, slo, shi = run_layer(hl, hh, ell, erl, elh, erh)
    hl, hh, ell, erl, elh, erh = _mid(
        olo[0], olo[1], ohi[0], ohi[1], slo[0], slo[1], shi[0], shi[1],
        E4, b0f[:, :_DP], b0f[:, _DP:], W1[:_DP], W1[_DP:], ALR1, 128)
    olo, ohi, slo, shi = run_layer(hl, hh, ell, erl, elh, erh)
    h2, el2, er2 = _mid(
        olo[0], olo[1], ohi[0], ohi[1], slo[0], slo[1], shi[0], shi[1],
        E4, b1f[:, :_DP], b1f[:, _DP:], W2p[:_DP], W2p[_DP:], ALR2, _DP)
    o2, s2 = _edge_pass(h2, el2, er2, src, dst)
    return _fin(o2[0], o2[1], s2[0], s2[1], b2f)
